# baseline (speedup 1.0000x reference)
import functools

import jax
import jax.numpy as jnp
from jax import lax
from jax.experimental import pallas as pl
from jax.experimental.pallas import tpu as pltpu
from jax.experimental.pallas import tpu_sc as plsc

N = 10000
D = 128
DH = 64
E = 320000
NC = 2
NS = 16
CHUNK = 80
EDGES_PER_TILE = E // NS
CHUNKS = EDGES_PER_TILE // CHUNK
NP = 10240
ROWS_PER_TILE = NP // NS
DEG_W = 16
NBUF = 7
GDEPTH = 5
HEAD = 5
PIECES = ROWS_PER_TILE // CHUNK


@functools.cache
def _make_sc_agg(with_deg: bool):
    out_type = [jax.ShapeDtypeStruct((NC, NP, DH), jnp.float32)]
    scratch = [
        pltpu.VMEM((CHUNKS, CHUNK), jnp.int32),
        pltpu.VMEM((CHUNKS, CHUNK), jnp.int32),
        pltpu.VMEM((NBUF, CHUNK, DH), jnp.float32),
        pltpu.VMEM_SHARED((NP, DH), jnp.float32),
    ] + [pltpu.SemaphoreType.DMA] * (2 * NBUF + 1)
    if with_deg:
        out_type.append(jax.ShapeDtypeStruct((NC, NP, DEG_W), jnp.float32))
        scratch += [
            pltpu.VMEM((CHUNK, DEG_W), jnp.float32),
            pltpu.VMEM((CHUNK, DEG_W), jnp.float32),
            pltpu.VMEM_SHARED((NP, DEG_W), jnp.float32),
        ]

    def body(*refs):
        if with_deg:
            (xs_hbm, src3_hbm, dst3_hbm, zrow_hbm, zdeg_hbm, one_hbm,
             acc_out, deg_out, sidx_all, didx_all, ring, acc_sh,
             *rest) = refs
            sems = rest[:2 * NBUF + 1]
            ones_v, degb, deg_sh = rest[2 * NBUF + 1:]
        else:
            (xs_hbm, src3_hbm, dst3_hbm, zrow_hbm,
             acc_out, sidx_all, didx_all, ring, acc_sh,
             *sems) = refs
        gsem = sems[:NBUF]
        ssem = sems[NBUF:2 * NBUF]
        dsem = sems[2 * NBUF]
        c = lax.axis_index("c")
        s = lax.axis_index("s")
        row0 = s * ROWS_PER_TILE

        pltpu.sync_copy(src3_hbm.at[c, s], sidx_all)
        pltpu.sync_copy(dst3_hbm.at[s], didx_all)
        pltpu.sync_copy(zrow_hbm, ring.at[0])
        for p in range(PIECES):
            pltpu.sync_copy(ring.at[0], acc_sh.at[pl.ds(row0 + p * CHUNK, CHUNK)])
        if with_deg:
            pltpu.sync_copy(one_hbm, ones_v)
            pltpu.sync_copy(zdeg_hbm, degb)
            for p in range(PIECES):
                pltpu.sync_copy(degb,
                                deg_sh.at[pl.ds(row0 + p * CHUNK, CHUNK)])
        plsc.subcore_barrier()


        def wait_gather(b):
            pltpu.make_async_copy(
                xs_hbm.at[pl.ds(0, CHUNK)], ring.at[b], gsem[b]).wait()

        def wait_scatter(b):
            pltpu.make_async_copy(
                zrow_hbm, ring.at[b], ssem[b]).wait()

        def wait_deg():
            pltpu.make_async_copy(zdeg_hbm, degb, dsem).wait()

        def fire_gather(j, b):
            pltpu.async_copy(xs_hbm.at[sidx_all.at[j]], ring.at[b], gsem[b])

        def step(j, b, deg_wait):
            wait_gather(b)
            pltpu.async_copy(ring.at[b], acc_sh.at[didx_all.at[j]], ssem[b],
                             add=True)
            if with_deg:
                @pl.when(lax.rem(j, 2) == c)
                def _():
                    if deg_wait:
                        wait_deg()
                    pltpu.async_copy(ones_v, deg_sh.at[didx_all.at[j]], dsem,
                                     add=True)

        for b in range(GDEPTH):
            fire_gather(b, b)
        for j0 in range(HEAD):
            step(j0, j0 % NBUF, deg_wait=(j0 >= 4))
            sn = (j0 + GDEPTH) % NBUF
            if j0 >= NBUF - GDEPTH:
                wait_scatter(sn)
            fire_gather(j0 + GDEPTH, sn)

        def pipe_body(jj, carry):
            for b in range(NBUF):
                j = HEAD + jj * NBUF + b
                slot = (HEAD + b) % NBUF
                step(j, slot, deg_wait=True)
                sn = (slot + GDEPTH) % NBUF
                wait_scatter(sn)

                @pl.when(j + GDEPTH < CHUNKS)
                def _():
                    fire_gather(j + GDEPTH, sn)
            return carry

        lax.fori_loop(0, (CHUNKS - HEAD) // NBUF, pipe_body, 0)
        wait_scatter((CHUNKS - 2) % NBUF)
        wait_scatter((CHUNKS - 1) % NBUF)
        if with_deg:
            for _ in range(2):
                wait_deg()
        plsc.subcore_barrier()

        for p in range(PIECES):
            piece = pl.ds(row0 + p * CHUNK, CHUNK)
            b = p % 2
            if p >= 2:
                pltpu.make_async_copy(
                    ring.at[b], acc_out.at[c, pl.ds(row0, CHUNK)],
                    gsem[b]).wait()
            pltpu.sync_copy(acc_sh.at[piece], ring.at[b])
            pltpu.async_copy(ring.at[b], acc_out.at[c, piece], gsem[b])
        for b in range(2):
            pltpu.make_async_copy(
                ring.at[b], acc_out.at[c, pl.ds(row0, CHUNK)], gsem[b]).wait()
        if with_deg:
            for p in range(PIECES):
                piece = pl.ds(row0 + p * CHUNK, CHUNK)
                pltpu.sync_copy(deg_sh.at[piece], degb)
                pltpu.sync_copy(degb, deg_out.at[c, piece])

    return pl.kernel(
        body,
        out_type=out_type,
        mesh=plsc.VectorSubcoreMesh(core_axis_name="c", subcore_axis_name="s",
                                    num_cores=NC, num_subcores=NS),
        scratch_types=scratch,
        compiler_params=pltpu.CompilerParams(use_tc_tiling_on_sc=False),
    )


BLK = 1000


def _tc_dense(xin, W, b):
    def body(x_ref, w_ref, b_ref, o_ref):
        o_ref[...] = (jnp.dot(x_ref[...], w_ref[...],
                              preferred_element_type=jnp.float32)
                      + b_ref[...])

    return pl.pallas_call(
        body,
        grid=(N // BLK,),
        in_specs=[
            pl.BlockSpec((BLK, D), lambda i: (i, 0)),
            pl.BlockSpec((D, D), lambda i: (0, 0)),
            pl.BlockSpec((1, D), lambda i: (0, 0)),
        ],
        out_specs=pl.BlockSpec((BLK, D), lambda i: (i, 0)),
        out_shape=jax.ShapeDtypeStruct((N, D), jnp.float32),
    )(xin, W, b.reshape(1, D))


def _tc_combine(acc, deg, dense, Wl, relu):
    def body(acc_ref, deg_ref, d_ref, wl_ref, o_ref):
        rd = 1.0 / jnp.maximum(deg_ref[0][:, :1] + deg_ref[1][:, :1], 1.0)
        wl = wl_ref[...]
        r = (jnp.dot(acc_ref[0] * rd, wl[:DH], preferred_element_type=jnp.float32)
             + jnp.dot(acc_ref[1] * rd, wl[DH:], preferred_element_type=jnp.float32)
             + d_ref[...])
        o_ref[...] = jnp.maximum(r, 0.0) if relu else r

    return pl.pallas_call(
        body,
        grid=(N // BLK,),
        in_specs=[
            pl.BlockSpec((NC, BLK, DH), lambda i: (0, i, 0)),
            pl.BlockSpec((NC, BLK, DEG_W), lambda i: (0, i, 0)),
            pl.BlockSpec((BLK, D), lambda i: (i, 0)),
            pl.BlockSpec((D, D), lambda i: (0, 0)),
        ],
        out_specs=pl.BlockSpec((BLK, D), lambda i: (i, 0)),
        out_shape=jax.ShapeDtypeStruct((N, D), jnp.float32),
    )(acc, deg, dense, Wl)


def kernel(x, edge_index, W1l, b1l, W1r, W2l, b2l, W2r):
    src = edge_index[0]
    dst = edge_index[1]
    src3 = jnp.concatenate([2 * src, 2 * src + 1]).reshape(NC, NS, CHUNKS, CHUNK)
    dst3 = dst.reshape(NS, CHUNKS, CHUNK)
    zrow = jnp.zeros((CHUNK, DH), jnp.float32)
    zdeg = jnp.zeros((CHUNK, DEG_W), jnp.float32)
    ones = jnp.ones((CHUNK, DEG_W), jnp.float32)

    dense1 = _tc_dense(x, W1r, b1l)
    acc1, deg = _make_sc_agg(True)(x.reshape(2 * N, DH), src3, dst3,
                                   zrow, zdeg, ones)
    h = _tc_combine(acc1, deg, dense1, W1l, relu=True)
    dense2 = _tc_dense(h, W2r, b2l)
    acc2 = _make_sc_agg(False)(h.reshape(2 * N, DH), src3, dst3, zrow)
    if isinstance(acc2, (list, tuple)):
        acc2 = acc2[0]
    out = _tc_combine(acc2, deg, dense2, W2l, relu=False)
    return out


# speedup vs baseline: 1.0312x; 1.0312x over previous
import functools

import jax
import jax.numpy as jnp
from jax import lax
from jax.experimental import pallas as pl
from jax.experimental.pallas import tpu as pltpu
from jax.experimental.pallas import tpu_sc as plsc

N = 10000
D = 128
DH = 64
E = 320000
NC = 2
NS = 16
CHUNK = 80
EDGES_PER_TILE = E // NS
CHUNKS = EDGES_PER_TILE // CHUNK
NP = 10240
ROWS_PER_TILE = NP // NS
DEG_W = 16
NBUF = 7
GDEPTH = 5
HEAD = 5
PIECES = ROWS_PER_TILE // CHUNK


@functools.cache
def _make_sc_agg(with_deg: bool):
    out_type = [jax.ShapeDtypeStruct((NC, NP, DH), jnp.float32)]
    scratch = [
        pltpu.VMEM((CHUNKS, CHUNK), jnp.int32),
        pltpu.VMEM((CHUNKS, CHUNK), jnp.int32),
        pltpu.VMEM((NBUF, CHUNK, DH), jnp.float32),
        pltpu.VMEM_SHARED((NP, DH), jnp.float32),
    ] + [pltpu.SemaphoreType.DMA] * (2 * NBUF + 1)
    if with_deg:
        out_type.append(jax.ShapeDtypeStruct((NC, NP, DEG_W), jnp.float32))
        scratch += [
            pltpu.VMEM((CHUNK, DEG_W), jnp.float32),
            pltpu.VMEM((CHUNK, DEG_W), jnp.float32),
            pltpu.VMEM_SHARED((NP, DEG_W), jnp.float32),
        ]

    def body(*refs):
        if with_deg:
            (xs_hbm, src3_hbm, dst3_hbm, zrow_hbm, zdeg_hbm, one_hbm,
             acc_out, deg_out, sidx_all, didx_all, ring, acc_sh,
             *rest) = refs
            sems = rest[:2 * NBUF + 1]
            ones_v, degb, deg_sh = rest[2 * NBUF + 1:]
        else:
            (xs_hbm, src3_hbm, dst3_hbm, zrow_hbm,
             acc_out, sidx_all, didx_all, ring, acc_sh,
             *sems) = refs
        gsem = sems[:NBUF]
        ssem = sems[NBUF:2 * NBUF]
        dsem = sems[2 * NBUF]
        c = lax.axis_index("c")
        s = lax.axis_index("s")
        row0 = s * ROWS_PER_TILE

        pltpu.sync_copy(src3_hbm.at[s], sidx_all)
        pltpu.sync_copy(dst3_hbm.at[s], didx_all)
        pltpu.sync_copy(zrow_hbm, ring.at[0])
        for p in range(PIECES):
            pltpu.sync_copy(ring.at[0], acc_sh.at[pl.ds(row0 + p * CHUNK, CHUNK)])
        if with_deg:
            pltpu.sync_copy(one_hbm, ones_v)
            pltpu.sync_copy(zdeg_hbm, degb)
            for p in range(PIECES):
                pltpu.sync_copy(degb,
                                deg_sh.at[pl.ds(row0 + p * CHUNK, CHUNK)])
        plsc.subcore_barrier()


        def wait_gather(b):
            pltpu.make_async_copy(
                xs_hbm.at[pl.ds(0, CHUNK)], ring.at[b], gsem[b]).wait()

        def wait_scatter(b):
            pltpu.make_async_copy(
                zrow_hbm, ring.at[b], ssem[b]).wait()

        def wait_deg():
            pltpu.make_async_copy(zdeg_hbm, degb, dsem).wait()

        def fire_gather(j, b):
            for k in range(CHUNK // 16):
                col = pl.ds(k * 16, 16)
                sidx_all[j, col] = sidx_all[j, col] * 2 + c
            pltpu.async_copy(xs_hbm.at[sidx_all.at[j]], ring.at[b], gsem[b])

        def step(j, b, deg_wait):
            wait_gather(b)
            pltpu.async_copy(ring.at[b], acc_sh.at[didx_all.at[j]], ssem[b],
                             add=True)
            if with_deg:
                @pl.when(lax.rem(j, 2) == c)
                def _():
                    if deg_wait:
                        wait_deg()
                    pltpu.async_copy(ones_v, deg_sh.at[didx_all.at[j]], dsem,
                                     add=True)

        for b in range(GDEPTH):
            fire_gather(b, b)
        for j0 in range(HEAD):
            step(j0, j0 % NBUF, deg_wait=(j0 >= 4))
            sn = (j0 + GDEPTH) % NBUF
            if j0 >= NBUF - GDEPTH:
                wait_scatter(sn)
            fire_gather(j0 + GDEPTH, sn)

        def pipe_body(jj, carry):
            for b in range(NBUF):
                j = HEAD + jj * NBUF + b
                slot = (HEAD + b) % NBUF
                step(j, slot, deg_wait=True)
                sn = (slot + GDEPTH) % NBUF
                wait_scatter(sn)

                @pl.when(j + GDEPTH < CHUNKS)
                def _():
                    fire_gather(j + GDEPTH, sn)
            return carry

        lax.fori_loop(0, (CHUNKS - HEAD) // NBUF, pipe_body, 0)
        wait_scatter((CHUNKS - 2) % NBUF)
        wait_scatter((CHUNKS - 1) % NBUF)
        if with_deg:
            for _ in range(2):
                wait_deg()
        plsc.subcore_barrier()

        for p in range(PIECES):
            piece = pl.ds(row0 + p * CHUNK, CHUNK)
            b = p % 2
            if p >= 2:
                pltpu.make_async_copy(
                    ring.at[b], acc_out.at[c, pl.ds(row0, CHUNK)],
                    gsem[b]).wait()
            pltpu.sync_copy(acc_sh.at[piece], ring.at[b])
            pltpu.async_copy(ring.at[b], acc_out.at[c, piece], gsem[b])
        for b in range(2):
            pltpu.make_async_copy(
                ring.at[b], acc_out.at[c, pl.ds(row0, CHUNK)], gsem[b]).wait()
        if with_deg:
            for p in range(PIECES):
                piece = pl.ds(row0 + p * CHUNK, CHUNK)
                pltpu.sync_copy(deg_sh.at[piece], degb)
                pltpu.sync_copy(degb, deg_out.at[c, piece])

    return pl.kernel(
        body,
        out_type=out_type,
        mesh=plsc.VectorSubcoreMesh(core_axis_name="c", subcore_axis_name="s",
                                    num_cores=NC, num_subcores=NS),
        scratch_types=scratch,
        compiler_params=pltpu.CompilerParams(use_tc_tiling_on_sc=False),
    )


BLK = 1000


def _tc_dense(xin, W, b):
    def body(x_ref, w_ref, b_ref, o_ref):
        o_ref[...] = (jnp.dot(x_ref[...], w_ref[...],
                              preferred_element_type=jnp.float32)
                      + b_ref[...])

    return pl.pallas_call(
        body,
        grid=(N // BLK,),
        in_specs=[
            pl.BlockSpec((BLK, D), lambda i: (i, 0)),
            pl.BlockSpec((D, D), lambda i: (0, 0)),
            pl.BlockSpec((1, D), lambda i: (0, 0)),
        ],
        out_specs=pl.BlockSpec((BLK, D), lambda i: (i, 0)),
        out_shape=jax.ShapeDtypeStruct((N, D), jnp.float32),
    )(xin, W, b.reshape(1, D))


def _tc_layer(acc, deg, xin, Wl, bl, Wr, relu):
    def body(acc_ref, deg_ref, x_ref, wl_ref, bl_ref, wr_ref, o_ref):
        rd = 1.0 / jnp.maximum(deg_ref[0][:, :1] + deg_ref[1][:, :1], 1.0)
        wl = wl_ref[...]
        r = (jnp.dot(acc_ref[0] * rd, wl[:DH], preferred_element_type=jnp.float32)
             + jnp.dot(acc_ref[1] * rd, wl[DH:], preferred_element_type=jnp.float32)
             + jnp.dot(x_ref[...], wr_ref[...], preferred_element_type=jnp.float32)
             + bl_ref[...])
        o_ref[...] = jnp.maximum(r, 0.0) if relu else r

    return pl.pallas_call(
        body,
        grid=(N // BLK,),
        in_specs=[
            pl.BlockSpec((NC, BLK, DH), lambda i: (0, i, 0)),
            pl.BlockSpec((NC, BLK, DEG_W), lambda i: (0, i, 0)),
            pl.BlockSpec((BLK, D), lambda i: (i, 0)),
            pl.BlockSpec((D, D), lambda i: (0, 0)),
            pl.BlockSpec((1, D), lambda i: (0, 0)),
            pl.BlockSpec((D, D), lambda i: (0, 0)),
        ],
        out_specs=pl.BlockSpec((BLK, D), lambda i: (i, 0)),
        out_shape=jax.ShapeDtypeStruct((N, D), jnp.float32),
    )(acc, deg, xin, Wl, bl.reshape(1, D), Wr)


def kernel(x, edge_index, W1l, b1l, W1r, W2l, b2l, W2r):
    src = edge_index[0]
    dst = edge_index[1]
    src3 = src.reshape(NS, CHUNKS, CHUNK)
    dst3 = dst.reshape(NS, CHUNKS, CHUNK)
    zrow = jnp.zeros((CHUNK, DH), jnp.float32)
    zdeg = jnp.zeros((CHUNK, DEG_W), jnp.float32)
    ones = jnp.ones((CHUNK, DEG_W), jnp.float32)

    acc1, deg = _make_sc_agg(True)(x.reshape(2 * N, DH), src3, dst3,
                                   zrow, zdeg, ones)
    h = _tc_layer(acc1, deg, x, W1l, b1l, W1r, relu=True)
    acc2 = _make_sc_agg(False)(h.reshape(2 * N, DH), src3, dst3, zrow)
    if isinstance(acc2, (list, tuple)):
        acc2 = acc2[0]
    out = _tc_layer(acc2, deg, h, W2l, b2l, W2r, relu=False)
    return out


# speedup vs baseline: 1.0747x; 1.0422x over previous
import functools

import jax
import jax.numpy as jnp
from jax import lax
from jax.experimental import pallas as pl
from jax.experimental.pallas import tpu as pltpu
from jax.experimental.pallas import tpu_sc as plsc

N = 10000
D = 128
DH = 64
E = 320000
NC = 2
NS = 16
CHUNK = 80
EDGES_PER_TILE = E // NS
CHUNKS = EDGES_PER_TILE // CHUNK
NP = 10240
ROWS_PER_TILE = NP // NS
DEG_W = 8
NBUF = 7
GDEPTH = 5
HEAD = 5
PIECES = ROWS_PER_TILE // CHUNK


@functools.cache
def _make_sc_agg(with_deg: bool):
    out_type = [jax.ShapeDtypeStruct((NC, NP, DH), jnp.float32)]
    scratch = [
        pltpu.VMEM((CHUNKS, CHUNK), jnp.int32),
        pltpu.VMEM((CHUNKS, CHUNK), jnp.int32),
        pltpu.VMEM((NBUF, CHUNK, DH), jnp.float32),
        pltpu.VMEM_SHARED((NP, DH), jnp.float32),
    ] + [pltpu.SemaphoreType.DMA] * (2 * NBUF + 1)
    if with_deg:
        out_type.append(jax.ShapeDtypeStruct((NC, NP, DEG_W), jnp.float32))
        scratch += [
            pltpu.VMEM((CHUNK, DEG_W), jnp.float32),
            pltpu.VMEM((CHUNK, DEG_W), jnp.float32),
            pltpu.VMEM_SHARED((NP, DEG_W), jnp.float32),
        ]

    def body(*refs):
        if with_deg:
            (xs_hbm, src3_hbm, dst3_hbm, zrow_hbm, zdeg_hbm, one_hbm,
             acc_out, deg_out, sidx_all, didx_all, ring, acc_sh,
             *rest) = refs
            sems = rest[:2 * NBUF + 1]
            ones_v, degb, deg_sh = rest[2 * NBUF + 1:]
        else:
            (xs_hbm, src3_hbm, dst3_hbm, zrow_hbm,
             acc_out, sidx_all, didx_all, ring, acc_sh,
             *sems) = refs
        gsem = sems[:NBUF]
        ssem = sems[NBUF:2 * NBUF]
        dsem = sems[2 * NBUF]
        c = lax.axis_index("c")
        s = lax.axis_index("s")
        row0 = s * ROWS_PER_TILE

        pltpu.sync_copy(src3_hbm.at[s], sidx_all)
        pltpu.sync_copy(dst3_hbm.at[s], didx_all)
        pltpu.sync_copy(zrow_hbm, ring.at[0])
        for p in range(PIECES):
            pltpu.sync_copy(ring.at[0], acc_sh.at[pl.ds(row0 + p * CHUNK, CHUNK)])
        if with_deg:
            pltpu.sync_copy(one_hbm, ones_v)
            pltpu.sync_copy(zdeg_hbm, degb)
            for p in range(PIECES):
                pltpu.sync_copy(degb,
                                deg_sh.at[pl.ds(row0 + p * CHUNK, CHUNK)])
        plsc.subcore_barrier()


        def wait_gather(b):
            pltpu.make_async_copy(
                xs_hbm.at[pl.ds(0, CHUNK)], ring.at[b], gsem[b]).wait()

        def wait_scatter(b):
            pltpu.make_async_copy(
                zrow_hbm, ring.at[b], ssem[b]).wait()

        def wait_deg():
            pltpu.make_async_copy(zdeg_hbm, degb, dsem).wait()

        def fire_gather(j, b):
            for k in range(CHUNK // 16):
                col = pl.ds(k * 16, 16)
                sidx_all[j, col] = sidx_all[j, col] * 2 + c
            pltpu.async_copy(xs_hbm.at[sidx_all.at[j]], ring.at[b], gsem[b])

        def step(j, b, deg_wait):
            wait_gather(b)
            pltpu.async_copy(ring.at[b], acc_sh.at[didx_all.at[j]], ssem[b],
                             add=True)
            if with_deg:
                @pl.when(lax.rem(j, 2) == c)
                def _():
                    if deg_wait:
                        wait_deg()
                    pltpu.async_copy(ones_v, deg_sh.at[didx_all.at[j]], dsem,
                                     add=True)

        for b in range(GDEPTH):
            fire_gather(b, b)
        for j0 in range(HEAD):
            step(j0, j0 % NBUF, deg_wait=(j0 >= 4))
            sn = (j0 + GDEPTH) % NBUF
            if j0 >= NBUF - GDEPTH:
                wait_scatter(sn)
            fire_gather(j0 + GDEPTH, sn)

        def pipe_body(jj, carry):
            for b in range(NBUF):
                j = HEAD + jj * NBUF + b
                slot = (HEAD + b) % NBUF
                step(j, slot, deg_wait=True)
                sn = (slot + GDEPTH) % NBUF
                wait_scatter(sn)

                @pl.when(j + GDEPTH < CHUNKS)
                def _():
                    fire_gather(j + GDEPTH, sn)
            return carry

        lax.fori_loop(0, (CHUNKS - HEAD) // NBUF, pipe_body, 0)
        wait_scatter((CHUNKS - 2) % NBUF)
        wait_scatter((CHUNKS - 1) % NBUF)
        if with_deg:
            for _ in range(2):
                wait_deg()
        plsc.subcore_barrier()

        for p in range(PIECES):
            piece = pl.ds(row0 + p * CHUNK, CHUNK)
            b = p % 2
            if p >= 2:
                pltpu.make_async_copy(
                    ring.at[b], acc_out.at[c, pl.ds(row0, CHUNK)],
                    gsem[b]).wait()
            pltpu.sync_copy(acc_sh.at[piece], ring.at[b])
            pltpu.async_copy(ring.at[b], acc_out.at[c, piece], gsem[b])
        for b in range(2):
            pltpu.make_async_copy(
                ring.at[b], acc_out.at[c, pl.ds(row0, CHUNK)], gsem[b]).wait()
        if with_deg:
            for p in range(PIECES):
                piece = pl.ds(row0 + p * CHUNK, CHUNK)
                pltpu.sync_copy(deg_sh.at[piece], degb)
                pltpu.sync_copy(degb, deg_out.at[c, piece])

    return pl.kernel(
        body,
        out_type=out_type,
        mesh=plsc.VectorSubcoreMesh(core_axis_name="c", subcore_axis_name="s",
                                    num_cores=NC, num_subcores=NS),
        scratch_types=scratch,
        compiler_params=pltpu.CompilerParams(use_tc_tiling_on_sc=False),
    )


BLK = 2000


def _tc_dense(xin, W, b):
    def body(x_ref, w_ref, b_ref, o_ref):
        o_ref[...] = (jnp.dot(x_ref[...], w_ref[...],
                              preferred_element_type=jnp.float32)
                      + b_ref[...])

    return pl.pallas_call(
        body,
        grid=(N // BLK,),
        in_specs=[
            pl.BlockSpec((BLK, D), lambda i: (i, 0)),
            pl.BlockSpec((D, D), lambda i: (0, 0)),
            pl.BlockSpec((1, D), lambda i: (0, 0)),
        ],
        out_specs=pl.BlockSpec((BLK, D), lambda i: (i, 0)),
        out_shape=jax.ShapeDtypeStruct((N, D), jnp.float32),
    )(xin, W, b.reshape(1, D))


def _tc_layer(acc, deg, xin, Wl, bl, Wr, relu):
    def body(acc_ref, deg_ref, x_ref, wl_ref, bl_ref, wr_ref, o_ref):
        rd = 1.0 / jnp.maximum(deg_ref[0][:, :1] + deg_ref[1][:, :1], 1.0)
        wl = wl_ref[...]
        r = (jnp.dot(acc_ref[0] * rd, wl[:DH], preferred_element_type=jnp.float32)
             + jnp.dot(acc_ref[1] * rd, wl[DH:], preferred_element_type=jnp.float32)
             + jnp.dot(x_ref[...], wr_ref[...], preferred_element_type=jnp.float32)
             + bl_ref[...])
        o_ref[...] = jnp.maximum(r, 0.0) if relu else r

    return pl.pallas_call(
        body,
        grid=(N // BLK,),
        in_specs=[
            pl.BlockSpec((NC, BLK, DH), lambda i: (0, i, 0)),
            pl.BlockSpec((NC, BLK, DEG_W), lambda i: (0, i, 0)),
            pl.BlockSpec((BLK, D), lambda i: (i, 0)),
            pl.BlockSpec((D, D), lambda i: (0, 0)),
            pl.BlockSpec((1, D), lambda i: (0, 0)),
            pl.BlockSpec((D, D), lambda i: (0, 0)),
        ],
        out_specs=pl.BlockSpec((BLK, D), lambda i: (i, 0)),
        out_shape=jax.ShapeDtypeStruct((N, D), jnp.float32),
    )(acc, deg, xin, Wl, bl.reshape(1, D), Wr)


def kernel(x, edge_index, W1l, b1l, W1r, W2l, b2l, W2r):
    src = edge_index[0]
    dst = edge_index[1]
    src3 = src.reshape(NS, CHUNKS, CHUNK)
    dst3 = dst.reshape(NS, CHUNKS, CHUNK)
    zrow = jnp.zeros((CHUNK, DH), jnp.float32)
    zdeg = jnp.zeros((CHUNK, DEG_W), jnp.float32)
    ones = jnp.ones((CHUNK, DEG_W), jnp.float32)

    acc1, deg = _make_sc_agg(True)(x.reshape(2 * N, DH), src3, dst3,
                                   zrow, zdeg, ones)
    h = _tc_layer(acc1, deg, x, W1l, b1l, W1r, relu=True)
    acc2 = _make_sc_agg(False)(h.reshape(2 * N, DH), src3, dst3, zrow)
    if isinstance(acc2, (list, tuple)):
        acc2 = acc2[0]
    out = _tc_layer(acc2, deg, h, W2l, b2l, W2r, relu=False)
    return out
